# Initial kernel scaffold; baseline (speedup 1.0000x reference)
import jax
import jax.numpy as jnp
from jax.experimental import pallas as pl


def kernel(z, W1, b1, W2, b2):
    raise NotImplementedError("write your pallas kernel here")


# speedup vs baseline: 1.5843x; 1.5843x over previous
import functools

import jax
import jax.numpy as jnp
from jax.experimental import pallas as pl
from jax.experimental.pallas import tpu as pltpu

N_PRIM = 64
VIEW = 4096
HIDDEN = 4096
TOPK = 8
TOKENS = 8192

M_TILE = 256
H_TILE = 512


def _router_kernel(z_ref, w1_ref, b1_ref, w2_ref, b2_ref, out_ref, acc_ref):
    h_idx = pl.program_id(0)
    n_h = pl.num_programs(0)
    m_idx = pl.program_id(1)
    rows = pl.ds(m_idx * M_TILE, M_TILE)

    h = jnp.dot(z_ref[...], w1_ref[...].T, preferred_element_type=jnp.float32)
    h = h + b1_ref[...]
    h = 0.5 * h * (1.0 + jax.lax.erf(h * 0.7071067811865476))
    partial = jnp.dot(h, w2_ref[...].T, preferred_element_type=jnp.float32)

    @pl.when(h_idx == 0)
    def _init():
        acc_ref[rows, :] = partial + b2_ref[...]

    @pl.when(h_idx != 0)
    def _accum():
        acc_ref[rows, :] = acc_ref[rows, :] + partial

    @pl.when(h_idx == n_h - 1)
    def _finalize():
        logits = acc_ref[rows, :]
        m = jnp.max(logits, axis=-1, keepdims=True)
        e = jnp.exp(logits - m)
        probs = e / jnp.sum(e, axis=-1, keepdims=True)
        lane = jax.lax.broadcasted_iota(jnp.int32, logits.shape, 1)
        cur = logits
        keep = jnp.zeros(logits.shape, dtype=jnp.bool_)
        for _ in range(TOPK):
            mx = jnp.max(cur, axis=-1, keepdims=True)
            is_mx = cur == mx
            first_lane = jnp.min(
                jnp.where(is_mx, lane, N_PRIM), axis=-1, keepdims=True
            )
            first = lane == first_lane
            keep = jnp.logical_or(keep, first)
            cur = jnp.where(first, -jnp.inf, cur)
        sparse = jnp.where(keep, probs, 0.0)
        denom = jnp.sum(sparse, axis=-1, keepdims=True) + 1e-8
        out_ref[...] = sparse / denom


@functools.partial(jax.jit, static_argnames=())
def kernel(z, W1, b1, W2, b2):
    n_h = HIDDEN // H_TILE
    n_m = TOKENS // M_TILE
    b1_2d = b1.reshape(1, HIDDEN)
    b2_2d = b2.reshape(1, N_PRIM)
    grid = (n_h, n_m)
    return pl.pallas_call(
        _router_kernel,
        grid=grid,
        in_specs=[
            pl.BlockSpec((M_TILE, VIEW), lambda h, m: (m, 0)),
            pl.BlockSpec((H_TILE, VIEW), lambda h, m: (h, 0)),
            pl.BlockSpec((1, H_TILE), lambda h, m: (0, h)),
            pl.BlockSpec((N_PRIM, H_TILE), lambda h, m: (0, h)),
            pl.BlockSpec((1, N_PRIM), lambda h, m: (0, 0)),
        ],
        out_specs=pl.BlockSpec((M_TILE, N_PRIM), lambda h, m: (m, 0)),
        out_shape=jax.ShapeDtypeStruct((TOKENS, N_PRIM), jnp.float32),
        scratch_shapes=[pltpu.VMEM((TOKENS, N_PRIM), jnp.float32)],
    )(z, W1, b1_2d, W2, b2_2d)


# speedup vs baseline: 2.0654x; 1.3036x over previous
import functools

import jax
import jax.numpy as jnp
from jax.experimental import pallas as pl
from jax.experimental.pallas import tpu as pltpu

N_PRIM = 64
VIEW = 4096
HIDDEN = 4096
TOPK = 8
TOKENS = 8192

M_TILE = 256
H_TILE = 1024


def _router_kernel(z_ref, w1_ref, b1_ref, w2_ref, b2_ref, out_ref, acc_ref):
    h_idx = pl.program_id(0)
    n_h = pl.num_programs(0)
    m_idx = pl.program_id(1)
    rows = pl.ds(m_idx * M_TILE, M_TILE)

    h = jnp.dot(z_ref[...], w1_ref[...].T, preferred_element_type=jnp.float32)
    h = h + b1_ref[...]
    h = 0.5 * h * (1.0 + jax.lax.erf(h * 0.7071067811865476))
    partial = jnp.dot(h, w2_ref[...].T, preferred_element_type=jnp.float32)

    @pl.when(h_idx == 0)
    def _init():
        acc_ref[rows, :] = partial + b2_ref[...]

    @pl.when(h_idx != 0)
    def _accum():
        acc_ref[rows, :] = acc_ref[rows, :] + partial

    @pl.when(h_idx == n_h - 1)
    def _finalize():
        logits = acc_ref[rows, :]
        m = jnp.max(logits, axis=-1, keepdims=True)
        e = jnp.exp(logits - m)
        probs = e / jnp.sum(e, axis=-1, keepdims=True)
        lane = jax.lax.broadcasted_iota(jnp.int32, logits.shape, 1)
        cur = logits
        keep = jnp.zeros(logits.shape, dtype=jnp.bool_)
        for _ in range(TOPK):
            mx = jnp.max(cur, axis=-1, keepdims=True)
            is_mx = cur == mx
            first_lane = jnp.min(
                jnp.where(is_mx, lane, N_PRIM), axis=-1, keepdims=True
            )
            first = lane == first_lane
            keep = jnp.logical_or(keep, first)
            cur = jnp.where(first, -jnp.inf, cur)
        sparse = jnp.where(keep, probs, 0.0)
        denom = jnp.sum(sparse, axis=-1, keepdims=True) + 1e-8
        out_ref[...] = sparse / denom


@functools.partial(jax.jit, static_argnames=())
def kernel(z, W1, b1, W2, b2):
    n_h = HIDDEN // H_TILE
    n_m = TOKENS // M_TILE
    b1_2d = b1.reshape(1, HIDDEN)
    b2_2d = b2.reshape(1, N_PRIM)
    grid = (n_h, n_m)
    return pl.pallas_call(
        _router_kernel,
        grid=grid,
        in_specs=[
            pl.BlockSpec((M_TILE, VIEW), lambda h, m: (m, 0)),
            pl.BlockSpec((H_TILE, VIEW), lambda h, m: (h, 0)),
            pl.BlockSpec((1, H_TILE), lambda h, m: (0, h)),
            pl.BlockSpec((N_PRIM, H_TILE), lambda h, m: (0, h)),
            pl.BlockSpec((1, N_PRIM), lambda h, m: (0, 0)),
        ],
        out_specs=pl.BlockSpec((M_TILE, N_PRIM), lambda h, m: (m, 0)),
        out_shape=jax.ShapeDtypeStruct((TOKENS, N_PRIM), jnp.float32),
        scratch_shapes=[pltpu.VMEM((TOKENS, N_PRIM), jnp.float32)],
    )(z, W1, b1_2d, W2, b2_2d)


# speedup vs baseline: 2.3782x; 1.1514x over previous
import functools

import jax
import jax.numpy as jnp
from jax.experimental import pallas as pl
from jax.experimental.pallas import tpu as pltpu

N_PRIM = 64
VIEW = 4096
HIDDEN = 4096
TOPK = 8
TOKENS = 8192

M_TILE = 512
H_TILE = 1024


def _router_kernel(z_ref, w1_ref, b1_ref, w2_ref, b2_ref, out_ref, acc_ref):
    h_idx = pl.program_id(0)
    n_h = pl.num_programs(0)
    m_idx = pl.program_id(1)
    rows = pl.ds(m_idx * M_TILE, M_TILE)

    h = jnp.dot(z_ref[...], w1_ref[...].T, preferred_element_type=jnp.float32)
    h = h + b1_ref[...]
    h = 0.5 * h * (1.0 + jax.lax.erf(h * 0.7071067811865476))
    partial = jnp.dot(h, w2_ref[...].T, preferred_element_type=jnp.float32)

    @pl.when(h_idx == 0)
    def _init():
        acc_ref[rows, :] = partial + b2_ref[...]

    @pl.when(h_idx != 0)
    def _accum():
        acc_ref[rows, :] = acc_ref[rows, :] + partial

    @pl.when(h_idx == n_h - 1)
    def _finalize():
        logits = acc_ref[rows, :]
        m = jnp.max(logits, axis=-1, keepdims=True)
        e = jnp.exp(logits - m)
        probs = e / jnp.sum(e, axis=-1, keepdims=True)
        lane = jax.lax.broadcasted_iota(jnp.int32, logits.shape, 1)
        cur = logits
        keep = jnp.zeros(logits.shape, dtype=jnp.bool_)
        for _ in range(TOPK):
            mx = jnp.max(cur, axis=-1, keepdims=True)
            is_mx = cur == mx
            first_lane = jnp.min(
                jnp.where(is_mx, lane, N_PRIM), axis=-1, keepdims=True
            )
            first = lane == first_lane
            keep = jnp.logical_or(keep, first)
            cur = jnp.where(first, -jnp.inf, cur)
        sparse = jnp.where(keep, probs, 0.0)
        denom = jnp.sum(sparse, axis=-1, keepdims=True) + 1e-8
        out_ref[...] = sparse / denom


@functools.partial(jax.jit, static_argnames=())
def kernel(z, W1, b1, W2, b2):
    n_h = HIDDEN // H_TILE
    n_m = TOKENS // M_TILE
    b1_2d = b1.reshape(1, HIDDEN)
    b2_2d = b2.reshape(1, N_PRIM)
    grid = (n_h, n_m)
    return pl.pallas_call(
        _router_kernel,
        grid=grid,
        in_specs=[
            pl.BlockSpec((M_TILE, VIEW), lambda h, m: (m, 0)),
            pl.BlockSpec((H_TILE, VIEW), lambda h, m: (h, 0)),
            pl.BlockSpec((1, H_TILE), lambda h, m: (0, h)),
            pl.BlockSpec((N_PRIM, H_TILE), lambda h, m: (0, h)),
            pl.BlockSpec((1, N_PRIM), lambda h, m: (0, 0)),
        ],
        out_specs=pl.BlockSpec((M_TILE, N_PRIM), lambda h, m: (m, 0)),
        out_shape=jax.ShapeDtypeStruct((TOKENS, N_PRIM), jnp.float32),
        scratch_shapes=[pltpu.VMEM((TOKENS, N_PRIM), jnp.float32)],
    )(z, W1, b1_2d, W2, b2_2d)
